# Initial kernel scaffold; baseline (speedup 1.0000x reference)
import jax
import jax.numpy as jnp
from jax.experimental import pallas as pl


def kernel(coords, features, neighbor_indices, W, b, gamma, beta, training):
    raise NotImplementedError("write your pallas kernel here")


# speedup vs baseline: 13.1905x; 13.1905x over previous
import functools
import math

import jax
import jax.numpy as jnp
import numpy as np
from jax import lax
from jax.experimental import pallas as pl
from jax.experimental.pallas import tpu as pltpu
from jax.experimental.pallas import tpu_sc as plsc

B, N, K, D, DIN, DOUT = 4, 8192, 16, 64, 10, 64
BN_EPS = 1e-6
SLOPE = 0.2

PB = 8
CH = PB * K
NTILES = 32
PPT = (B * N) // NTILES
NCHUNK = PPT // PB
E = B * N * K


def _tc_pre(coords2, W, b, gamma, beta):
    BLK = 4096
    grid = (B * N) // BLK
    inv = np.float32(1.0 / math.sqrt(1.0 + BN_EPS))

    def body(c_ref, w_ref, b_ref, g_ref, be_ref, p2_ref, q1_ref, w0_ref):
        w = w_ref[...]
        s = g_ref[...] * inv
        c = c_ref[...]
        p2 = jnp.zeros((BLK, DOUT), jnp.float32)
        q1 = jnp.zeros((BLK, DOUT), jnp.float32)
        for dim in range(3):
            a_row = w[1 + dim:2 + dim, :] + w[4 + dim:5 + dim, :]
            c_row = w[7 + dim:8 + dim, :] - w[1 + dim:2 + dim, :]
            cd = c[:, dim:dim + 1]
            p2 = p2 + cd * a_row
            q1 = q1 + cd * c_row
        p2_ref[...] = (p2 + b_ref[...]) * s + be_ref[...]
        q1_ref[...] = q1 * s
        w0_ref[...] = w[0:1, :] * s

    return pl.pallas_call(
        body,
        grid=(grid,),
        in_specs=[
            pl.BlockSpec((BLK, 3), lambda i: (i, 0)),
            pl.BlockSpec((DIN, DOUT), lambda i: (0, 0)),
            pl.BlockSpec((1, DOUT), lambda i: (0, 0)),
            pl.BlockSpec((1, DOUT), lambda i: (0, 0)),
            pl.BlockSpec((1, DOUT), lambda i: (0, 0)),
        ],
        out_specs=[
            pl.BlockSpec((BLK, DOUT), lambda i: (i, 0)),
            pl.BlockSpec((BLK, DOUT), lambda i: (i, 0)),
            pl.BlockSpec((1, DOUT), lambda i: (0, 0)),
        ],
        out_shape=[
            jax.ShapeDtypeStruct((B * N, DOUT), jnp.float32),
            jax.ShapeDtypeStruct((B * N, DOUT), jnp.float32),
            jax.ShapeDtypeStruct((1, DOUT), jnp.float32),
        ],
    )(coords2, W, b.reshape(1, DOUT), gamma.reshape(1, DOUT),
      beta.reshape(1, DOUT))


def _i16(v):
    return jnp.full((16,), v, jnp.int32)


def _sc_main(coords_t, idx_flat, feats2, p2, q1, w0s):
    mesh = plsc.VectorSubcoreMesh(core_axis_name="c", subcore_axis_name="s")

    @functools.partial(
        pl.kernel,
        mesh=mesh,
        compiler_params=pltpu.CompilerParams(
            needs_layout_passes=False, use_tc_tiling_on_sc=False),
        out_type=[
            jax.ShapeDtypeStruct((2 * E, DOUT), jnp.float32),
            jax.ShapeDtypeStruct((E, DOUT), jnp.float32),
        ],
        scratch_types=[
            pltpu.VMEM((N,), jnp.float32),
            pltpu.VMEM((N,), jnp.float32),
            pltpu.VMEM((N,), jnp.float32),
            pltpu.VMEM((CH,), jnp.int32),
            pltpu.VMEM((CH,), jnp.int32),
            pltpu.VMEM((CH,), jnp.int32),
            pltpu.VMEM((CH,), jnp.int32),
            pltpu.VMEM((CH, DOUT), jnp.float32),
            pltpu.VMEM((CH, DOUT), jnp.float32),
            pltpu.VMEM((CH, DOUT), jnp.float32),
            pltpu.VMEM((PB, DOUT), jnp.float32),
            pltpu.VMEM((DOUT,), jnp.float32),
            pltpu.SemaphoreType.DMA,
            pltpu.SemaphoreType.DMA,
        ],
    )
    def k(coords_hbm, idx_hbm, feats_hbm, p2_hbm, q1_hbm, w0_hbm,
          outv_hbm, relv_hbm,
          cplx_v, cply_v, cplz_v, lidx_v, gidx_v, eidx_v, oidx_v,
          qbuf, fbuf, relbuf, pbuf, w0_v, sem1, sem2):
        wid = lax.axis_index("s") * 2 + lax.axis_index("c")
        batch = wid // (NTILES // B)
        pbase0 = wid * PPT
        pltpu.sync_copy(coords_hbm.at[batch * 3 + 0], cplx_v)
        pltpu.sync_copy(coords_hbm.at[batch * 3 + 1], cply_v)
        pltpu.sync_copy(coords_hbm.at[batch * 3 + 2], cplz_v)
        pltpu.sync_copy(w0_hbm.at[0], w0_v)

        def chunk_body(ci, carry):
            pbase = pbase0 + ci * PB
            ebase = pbase * K
            pltpu.sync_copy(idx_hbm.at[pl.ds(ebase, CH)], lidx_v)
            pltpu.sync_copy(p2_hbm.at[pl.ds(pbase, PB)], pbuf)
            off = batch * N
            for i in range(CH // 16):
                sl = pl.ds(i * 16, 16)
                gidx_v[sl] = lidx_v[sl] + off
                er = (lax.iota(jnp.int32, 16) + (ebase + i * 16)) * 2
                eidx_v[sl] = er
                oidx_v[sl] = er + 1
            cp1 = pltpu.async_copy(q1_hbm.at[gidx_v], qbuf, sem1)
            cp2 = pltpu.async_copy(feats_hbm.at[gidx_v], fbuf, sem2)
            cp1.wait()
            cp2.wait()

            def point_body(p, c2):
                lp = (pbase - off) + p
                nidx = lidx_v[pl.ds(p * K, 16)]
                nx = plsc.load_gather(cplx_v, [nidx])
                ny = plsc.load_gather(cply_v, [nidx])
                nz = plsc.load_gather(cplz_v, [nidx])
                lpv = jnp.full((16,), lp, jnp.int32)
                px = plsc.load_gather(cplx_v, [lpv])
                py = plsc.load_gather(cply_v, [lpv])
                pz = plsc.load_gather(cplz_v, [lpv])
                dx = px - nx
                dy = py - ny
                dz = pz - nz
                d2 = dx * dx + dy * dy + dz * dz + jnp.float32(1e-12)
                ib = plsc.bitcast(d2, jnp.int32)
                r = plsc.bitcast(jnp.int32(0x5F3759DF) - (ib >> 1),
                                 jnp.float32)
                half = d2 * jnp.float32(0.5)
                for _ in range(3):
                    r = r * (jnp.float32(1.5) - half * r * r)
                dist = d2 * r
                for kk in range(K):
                    dsp = jnp.full((16,), dist[kk], jnp.float32)
                    erow = p * K + kk
                    for c4 in range(4):
                        sl = pl.ds(c4 * 16, 16)
                        y = (pbuf[p, sl] + qbuf[erow, sl]
                             + dsp * w0_v[sl])
                        relbuf[erow, sl] = jnp.maximum(
                            y, y * jnp.float32(SLOPE))
                return c2

            lax.fori_loop(0, PB, point_body, 0)
            pltpu.sync_copy(relbuf, relv_hbm.at[pl.ds(ebase, CH)])
            cp3 = pltpu.async_copy(fbuf, outv_hbm.at[eidx_v], sem1)
            cp4 = pltpu.async_copy(relbuf, outv_hbm.at[oidx_v], sem2)
            cp3.wait()
            cp4.wait()
            return carry

        lax.fori_loop(0, NCHUNK, chunk_body, 0)

    return k(coords_t, idx_flat, feats2, p2, q1, w0s)


def kernel(coords, features, neighbor_indices, W, b, gamma, beta, training):
    coords2 = coords.reshape(B * N, 3)
    coords_t = jnp.transpose(coords, (0, 2, 1)).reshape(B * 3, N)
    feats2 = features.reshape(B * N, D)
    idx_flat = neighbor_indices.reshape(-1)
    p2, q1, w0s = _tc_pre(coords2, W, b, gamma, beta)
    outv, relv = _sc_main(coords_t, idx_flat, feats2, p2, q1, w0s)
    out = outv.reshape(B, N, K, 2 * DOUT)
    rel = relv.reshape(B, N, K, DOUT)
    return (out, rel)


# speedup vs baseline: 16.7304x; 1.2684x over previous
import functools
import math

import jax
import jax.numpy as jnp
import numpy as np
from jax import lax
from jax.experimental import pallas as pl
from jax.experimental.pallas import tpu as pltpu
from jax.experimental.pallas import tpu_sc as plsc

B, N, K, D, DIN, DOUT = 4, 8192, 16, 64, 10, 64
BN_EPS = 1e-6
SLOPE = 0.2

PB = 8
CH = PB * K
NTILES = 32
PPT = (B * N) // NTILES
NCHUNK = PPT // PB
SETS = 4
ROUNDS = NCHUNK // SETS
E = B * N * K


def _tc_pre(coords2, feats2, W, b, gamma, beta):
    BLK = 4096
    grid = (B * N) // BLK
    inv = np.float32(1.0 / math.sqrt(1.0 + BN_EPS))

    def body(c_ref, f_ref, w_ref, b_ref, g_ref, be_ref,
             fq_ref, p2_ref, w0_ref):
        w = w_ref[...]
        s = g_ref[...] * inv
        c = c_ref[...]
        p2 = jnp.zeros((BLK, DOUT), jnp.float32)
        q1 = jnp.zeros((BLK, DOUT), jnp.float32)
        for dim in range(3):
            a_row = w[1 + dim:2 + dim, :] + w[4 + dim:5 + dim, :]
            c_row = w[7 + dim:8 + dim, :] - w[1 + dim:2 + dim, :]
            cd = c[:, dim:dim + 1]
            p2 = p2 + cd * a_row
            q1 = q1 + cd * c_row
        fq_ref[...] = jnp.concatenate([f_ref[...], q1 * s], axis=-1)
        p2_ref[...] = (p2 + b_ref[...]) * s + be_ref[...]
        w0_ref[...] = w[0:1, :] * s

    return pl.pallas_call(
        body,
        grid=(grid,),
        in_specs=[
            pl.BlockSpec((BLK, 3), lambda i: (i, 0)),
            pl.BlockSpec((BLK, D), lambda i: (i, 0)),
            pl.BlockSpec((DIN, DOUT), lambda i: (0, 0)),
            pl.BlockSpec((1, DOUT), lambda i: (0, 0)),
            pl.BlockSpec((1, DOUT), lambda i: (0, 0)),
            pl.BlockSpec((1, DOUT), lambda i: (0, 0)),
        ],
        out_specs=[
            pl.BlockSpec((BLK, D + DOUT), lambda i: (i, 0)),
            pl.BlockSpec((BLK, DOUT), lambda i: (i, 0)),
            pl.BlockSpec((1, DOUT), lambda i: (0, 0)),
        ],
        out_shape=[
            jax.ShapeDtypeStruct((B * N, D + DOUT), jnp.float32),
            jax.ShapeDtypeStruct((B * N, DOUT), jnp.float32),
            jax.ShapeDtypeStruct((1, DOUT), jnp.float32),
        ],
    )(coords2, feats2, W, b.reshape(1, DOUT), gamma.reshape(1, DOUT),
      beta.reshape(1, DOUT))


def _sc_main(coords_t, idx_flat, fq, p2, w0s):
    mesh = plsc.VectorSubcoreMesh(core_axis_name="c", subcore_axis_name="s")

    @functools.partial(
        pl.kernel,
        mesh=mesh,
        compiler_params=pltpu.CompilerParams(
            needs_layout_passes=False, use_tc_tiling_on_sc=False),
        out_type=[
            jax.ShapeDtypeStruct((E, 2 * DOUT), jnp.float32),
            jax.ShapeDtypeStruct((E, DOUT), jnp.float32),
        ],
        scratch_types=[
            pltpu.VMEM((N,), jnp.float32),
            pltpu.VMEM((N,), jnp.float32),
            pltpu.VMEM((N,), jnp.float32),
            pltpu.VMEM((PPT * K,), jnp.int32),
            pltpu.VMEM((DOUT,), jnp.float32),
        ] + [pltpu.VMEM((CH, 2 * DOUT), jnp.float32) for _ in range(SETS)]
          + [pltpu.VMEM((PB, DOUT), jnp.float32) for _ in range(SETS)]
          + [pltpu.SemaphoreType.DMA for _ in range(2 * SETS)],
    )
    def k(coords_hbm, idx_hbm, fq_hbm, p2_hbm, w0_hbm,
          outv_hbm, relv_hbm,
          cplx_v, cply_v, cplz_v, idx_all, w0_v,
          ob0, ob1, ob2, ob3, pb0, pb1, pb2, pb3,
          gs0, gs1, gs2, gs3, ss0, ss1, ss2, ss3):
        obufs = [ob0, ob1, ob2, ob3]
        pbufs = [pb0, pb1, pb2, pb3]
        gsems = [gs0, gs1, gs2, gs3]
        ssems = [ss0, ss1, ss2, ss3]
        wid = lax.axis_index("s") * 2 + lax.axis_index("c")
        batch = wid // (NTILES // B)
        pbase0 = wid * PPT
        lbase0 = pbase0 - batch * N
        ebase0 = pbase0 * K
        pltpu.sync_copy(coords_hbm.at[batch * 3 + 0], cplx_v)
        pltpu.sync_copy(coords_hbm.at[batch * 3 + 1], cply_v)
        pltpu.sync_copy(coords_hbm.at[batch * 3 + 2], cplz_v)
        pltpu.sync_copy(w0_hbm.at[0], w0_v)
        pltpu.sync_copy(idx_hbm.at[pl.ds(ebase0, PPT * K)], idx_all)
        fq_b = fq_hbm.at[batch]

        def gather_cps(c, s):
            src = fq_b.at[idx_all.at[pl.ds(c * CH, CH)]]
            cp1 = pltpu.make_async_copy(src, obufs[s], gsems[s])
            cp2 = pltpu.make_async_copy(
                p2_hbm.at[pl.ds(pbase0 + c * PB, PB)], pbufs[s], gsems[s])
            return cp1, cp2

        def scatter_cps(c, s):
            eb = ebase0 + c * CH
            cp1 = pltpu.make_async_copy(
                obufs[s], outv_hbm.at[pl.ds(eb, CH)], ssems[s])
            cp2 = pltpu.make_async_copy(
                obufs[s].at[:, pl.ds(DOUT, DOUT)],
                relv_hbm.at[pl.ds(eb, CH)], ssems[s])
            return cp1, cp2

        def pre(c, s):
            cp1, cp2 = gather_cps(c, s)
            cp1.start()
            cp2.start()

        def wait_gather(c, s):
            cp1, cp2 = gather_cps(c, s)
            cp1.wait()
            cp2.wait()

        def fire_scatter(c, s):
            cp1, cp2 = scatter_cps(c, s)
            cp1.start()
            cp2.start()

        def wait_scatter(c, s):
            cp1, cp2 = scatter_cps(c, s)
            cp1.wait()
            cp2.wait()

        def compute(c, s):
            obuf = obufs[s]
            pbuf = pbufs[s]

            def point_body(p, carry):
                lp = lbase0 + c * PB + p
                nidx = idx_all[pl.ds((c * PB + p) * K, 16)]
                nx = plsc.load_gather(cplx_v, [nidx])
                ny = plsc.load_gather(cply_v, [nidx])
                nz = plsc.load_gather(cplz_v, [nidx])
                lpv = jnp.full((16,), lp, jnp.int32)
                px = plsc.load_gather(cplx_v, [lpv])
                py = plsc.load_gather(cply_v, [lpv])
                pz = plsc.load_gather(cplz_v, [lpv])
                dx = px - nx
                dy = py - ny
                dz = pz - nz
                d2 = dx * dx + dy * dy + dz * dz + jnp.float32(1e-12)
                ib = plsc.bitcast(d2, jnp.int32)
                r = plsc.bitcast(jnp.int32(0x5F3759DF) - (ib >> 1),
                                 jnp.float32)
                half = d2 * jnp.float32(0.5)
                for _ in range(3):
                    r = r * (jnp.float32(1.5) - half * r * r)
                dist = d2 * r
                for kk in range(K):
                    dsp = jnp.full((16,), dist[kk], jnp.float32)
                    erow = p * K + kk
                    for c4 in range(4):
                        sl = pl.ds(c4 * 16, 16)
                        osl = pl.ds(DOUT + c4 * 16, 16)
                        y = (pbuf[p, sl] + obuf[erow, osl]
                             + dsp * w0_v[sl])
                        obuf[erow, osl] = jnp.maximum(
                            y, y * jnp.float32(SLOPE))
                return carry

            lax.fori_loop(0, PB, point_body, 0)

        pre(0, 0)
        pre(1, 1)

        def round_body(rr, carry):
            for bb in range(SETS):
                g = rr * SETS + bb
                s2 = (bb + 2) % SETS
                if bb >= 2:
                    wait_scatter(g - 2, s2)
                else:
                    @pl.when(rr >= 1)
                    def _drain():
                        wait_scatter(g - 2, s2)
                pre(jnp.minimum(g + 2, NCHUNK - 1), s2)
                wait_gather(g, bb)
                compute(g, bb)
                fire_scatter(g, bb)
            return carry

        lax.fori_loop(0, ROUNDS, round_body, 0)
        wait_scatter(NCHUNK - 2, (NCHUNK - 2) % SETS)
        wait_scatter(NCHUNK - 1, (NCHUNK - 1) % SETS)
        wait_gather(NCHUNK - 1, 0)
        wait_gather(NCHUNK - 1, 1)

    return k(coords_t, idx_flat, fq, p2, w0s)


def kernel(coords, features, neighbor_indices, W, b, gamma, beta, training):
    coords2 = coords.reshape(B * N, 3)
    coords_t = jnp.transpose(coords, (0, 2, 1)).reshape(B * 3, N)
    feats2 = features.reshape(B * N, D)
    idx_flat = neighbor_indices.reshape(-1)
    fq, p2, w0s = _tc_pre(coords2, feats2, W, b, gamma, beta)
    outv, relv = _sc_main(coords_t, idx_flat, fq.reshape(B, N, D + DOUT),
                          p2, w0s)
    out = outv.reshape(B, N, K, 2 * DOUT)
    rel = relv.reshape(B, N, K, DOUT)
    return (out, rel)


# speedup vs baseline: 28.3986x; 1.6974x over previous
import functools
import math

import jax
import jax.numpy as jnp
import numpy as np
from jax import lax
from jax.experimental import pallas as pl
from jax.experimental.pallas import tpu as pltpu
from jax.experimental.pallas import tpu_sc as plsc

B, N, K, D, DIN, DOUT = 4, 8192, 16, 64, 10, 64
BN_EPS = 1e-6
SLOPE = 0.2

PB = 8
CH = PB * K
NTILES = 32
PPT = (B * N) // NTILES
NCHUNK = PPT // PB
SETS = 4
ROUNDS = NCHUNK // SETS
E = B * N * K


def _tc_pre(coords2, feats2, W, b, gamma, beta):
    BLK = 4096
    grid = (B * N) // BLK
    inv = np.float32(1.0 / math.sqrt(1.0 + BN_EPS))

    def body(c_ref, f_ref, w_ref, b_ref, g_ref, be_ref,
             fq_ref, p2_ref, w0_ref):
        w = w_ref[...]
        s = g_ref[...] * inv
        c = c_ref[...]
        p2 = jnp.zeros((BLK, DOUT), jnp.float32)
        q1 = jnp.zeros((BLK, DOUT), jnp.float32)
        for dim in range(3):
            a_row = w[1 + dim:2 + dim, :] + w[4 + dim:5 + dim, :]
            c_row = w[7 + dim:8 + dim, :] - w[1 + dim:2 + dim, :]
            cd = c[:, dim:dim + 1]
            p2 = p2 + cd * a_row
            q1 = q1 + cd * c_row
        fq_ref[...] = jnp.concatenate([f_ref[...], q1 * s], axis=-1)
        p2_ref[...] = (p2 + b_ref[...]) * s + be_ref[...]
        w0_ref[...] = w[0:1, :] * s

    return pl.pallas_call(
        body,
        grid=(grid,),
        in_specs=[
            pl.BlockSpec((BLK, 3), lambda i: (i, 0)),
            pl.BlockSpec((BLK, D), lambda i: (i, 0)),
            pl.BlockSpec((DIN, DOUT), lambda i: (0, 0)),
            pl.BlockSpec((1, DOUT), lambda i: (0, 0)),
            pl.BlockSpec((1, DOUT), lambda i: (0, 0)),
            pl.BlockSpec((1, DOUT), lambda i: (0, 0)),
        ],
        out_specs=[
            pl.BlockSpec((BLK, D + DOUT), lambda i: (i, 0)),
            pl.BlockSpec((BLK, DOUT), lambda i: (i, 0)),
            pl.BlockSpec((1, DOUT), lambda i: (0, 0)),
        ],
        out_shape=[
            jax.ShapeDtypeStruct((B * N, D + DOUT), jnp.float32),
            jax.ShapeDtypeStruct((B * N, DOUT), jnp.float32),
            jax.ShapeDtypeStruct((1, DOUT), jnp.float32),
        ],
    )(coords2, feats2, W, b.reshape(1, DOUT), gamma.reshape(1, DOUT),
      beta.reshape(1, DOUT))


def _sc_main(coords_t, idx_flat, fq, p2, w0s):
    mesh = plsc.VectorSubcoreMesh(core_axis_name="c", subcore_axis_name="s")

    @functools.partial(
        pl.kernel,
        mesh=mesh,
        compiler_params=pltpu.CompilerParams(
            needs_layout_passes=False, use_tc_tiling_on_sc=False),
        out_type=[
            jax.ShapeDtypeStruct((E, 2 * DOUT), jnp.float32),
            jax.ShapeDtypeStruct((E, DOUT), jnp.float32),
        ],
        scratch_types=[
            pltpu.VMEM((N,), jnp.float32),
            pltpu.VMEM((N,), jnp.float32),
            pltpu.VMEM((N,), jnp.float32),
            pltpu.VMEM((PPT * K,), jnp.int32),
            pltpu.VMEM((DOUT,), jnp.float32),
        ] + [pltpu.VMEM((CH, 2 * DOUT), jnp.float32) for _ in range(SETS)]
          + [pltpu.VMEM((PB, DOUT), jnp.float32) for _ in range(SETS)]
          + [pltpu.SemaphoreType.DMA for _ in range(2 * SETS)],
    )
    def k(coords_hbm, idx_hbm, fq_hbm, p2_hbm, w0_hbm,
          outv_hbm, relv_hbm,
          cplx_v, cply_v, cplz_v, idx_all, w0_v,
          ob0, ob1, ob2, ob3, pb0, pb1, pb2, pb3,
          gs0, gs1, gs2, gs3, ss0, ss1, ss2, ss3):
        obufs = [ob0, ob1, ob2, ob3]
        pbufs = [pb0, pb1, pb2, pb3]
        gsems = [gs0, gs1, gs2, gs3]
        ssems = [ss0, ss1, ss2, ss3]
        wid = lax.axis_index("s") * 2 + lax.axis_index("c")
        batch = wid // (NTILES // B)
        pbase0 = wid * PPT
        lbase0 = pbase0 - batch * N
        ebase0 = pbase0 * K
        pltpu.sync_copy(coords_hbm.at[batch * 3 + 0], cplx_v)
        pltpu.sync_copy(coords_hbm.at[batch * 3 + 1], cply_v)
        pltpu.sync_copy(coords_hbm.at[batch * 3 + 2], cplz_v)
        pltpu.sync_copy(w0_hbm.at[0], w0_v)
        pltpu.sync_copy(idx_hbm.at[pl.ds(ebase0, PPT * K)], idx_all)
        fq_b = fq_hbm.at[batch]

        def gather_cps(c, s):
            src = fq_b.at[idx_all.at[pl.ds(c * CH, CH)]]
            cp1 = pltpu.make_async_copy(src, obufs[s], gsems[s])
            cp2 = pltpu.make_async_copy(
                p2_hbm.at[pl.ds(pbase0 + c * PB, PB)], pbufs[s], gsems[s])
            return cp1, cp2

        def scatter_cps(c, s):
            eb = ebase0 + c * CH
            cp1 = pltpu.make_async_copy(
                obufs[s], outv_hbm.at[pl.ds(eb, CH)], ssems[s])
            cp2 = pltpu.make_async_copy(
                obufs[s].at[:, pl.ds(DOUT, DOUT)],
                relv_hbm.at[pl.ds(eb, CH)], ssems[s])
            return cp1, cp2

        def pre(c, s):
            cp1, cp2 = gather_cps(c, s)
            cp1.start()
            cp2.start()

        def wait_gather(c, s):
            cp1, cp2 = gather_cps(c, s)
            cp1.wait()
            cp2.wait()

        def fire_scatter(c, s):
            cp1, cp2 = scatter_cps(c, s)
            cp1.start()
            cp2.start()

        def wait_scatter(c, s):
            cp1, cp2 = scatter_cps(c, s)
            cp1.wait()
            cp2.wait()

        def compute(c, s):
            obuf = obufs[s]
            pbuf = pbufs[s]

            def point_body(p, carry):
                lp = lbase0 + c * PB + p
                nidx = idx_all[pl.ds((c * PB + p) * K, 16)]
                nx = plsc.load_gather(cplx_v, [nidx])
                ny = plsc.load_gather(cply_v, [nidx])
                nz = plsc.load_gather(cplz_v, [nidx])
                lpv = jnp.full((16,), lp, jnp.int32)
                px = plsc.load_gather(cplx_v, [lpv])
                py = plsc.load_gather(cply_v, [lpv])
                pz = plsc.load_gather(cplz_v, [lpv])
                dx = px - nx
                dy = py - ny
                dz = pz - nz
                d2 = dx * dx + dy * dy + dz * dz + jnp.float32(1e-12)
                ib = plsc.bitcast(d2, jnp.int32)
                r = plsc.bitcast(jnp.int32(0x5F3759DF) - (ib >> 1),
                                 jnp.float32)
                half = d2 * jnp.float32(0.5)
                for _ in range(3):
                    r = r * (jnp.float32(1.5) - half * r * r)
                dist = d2 * r
                pv = [pbuf[p, pl.ds(c4 * 16, 16)] for c4 in range(4)]
                w0r = [w0_v[pl.ds(c4 * 16, 16)] for c4 in range(4)]
                for kk in range(K):
                    dsp = jnp.full((16,), dist[kk], jnp.float32)
                    erow = p * K + kk
                    for c4 in range(4):
                        osl = pl.ds(DOUT + c4 * 16, 16)
                        y = (pv[c4] + dsp * w0r[c4]) + obuf[erow, osl]
                        obuf[erow, osl] = jnp.maximum(
                            y, y * jnp.float32(SLOPE))
                return carry

            lax.fori_loop(0, PB, point_body, 0)

        pre(0, 0)
        pre(1, 1)

        def round_body(rr, carry):
            for bb in range(SETS):
                g = rr * SETS + bb
                s2 = (bb + 2) % SETS
                if bb >= 2:
                    wait_scatter(g - 2, s2)
                else:
                    @pl.when(rr >= 1)
                    def _drain():
                        wait_scatter(g - 2, s2)
                pre(jnp.minimum(g + 2, NCHUNK - 1), s2)
                wait_gather(g, bb)
                compute(g, bb)
                fire_scatter(g, bb)
            return carry

        lax.fori_loop(0, ROUNDS, round_body, 0)
        wait_scatter(NCHUNK - 2, (NCHUNK - 2) % SETS)
        wait_scatter(NCHUNK - 1, (NCHUNK - 1) % SETS)
        wait_gather(NCHUNK - 1, 0)
        wait_gather(NCHUNK - 1, 1)

    return k(coords_t, idx_flat, fq, p2, w0s)


def kernel(coords, features, neighbor_indices, W, b, gamma, beta, training):
    coords2 = coords.reshape(B * N, 3)
    coords_t = jnp.transpose(coords, (0, 2, 1)).reshape(B * 3, N)
    feats2 = features.reshape(B * N, D)
    idx_flat = neighbor_indices.reshape(-1)
    fq, p2, w0s = _tc_pre(coords2, feats2, W, b, gamma, beta)
    outv, relv = _sc_main(coords_t, idx_flat, fq.reshape(B, N, D + DOUT),
                          p2, w0s)
    out = outv.reshape(B, N, K, 2 * DOUT)
    rel = relv.reshape(B, N, K, DOUT)
    return (out, rel)
